# baseline (speedup 1.0000x reference)
import functools

import jax
import jax.numpy as jnp
from jax import lax
from jax.experimental import pallas as pl
from jax.experimental.pallas import tpu as pltpu
from jax.experimental.pallas import tpu_sc as plsc

B, T, D = 4096, 200, 64
NW = 32
TG = T // 8


def _body(x4_hbm, tab_hbm, pos_hbm, out_hbm, idx_v, pos_v, bufg0, bufg1,
          bufo0, bufo1, gsem, wsem0, wsem1):
    wid = lax.axis_index("s") * 2 + lax.axis_index("c")

    pltpu.sync_copy(x4_hbm.at[wid], idx_v)
    pltpu.sync_copy(pos_hbm, pos_v)

    k = lax.iota(jnp.int32, 16)
    rv = lax.bitwise_and(k, 7)
    dgv = lax.shift_right_logical(k, 3)

    bufgs = (bufg0, bufg1)
    bufos = (bufo0, bufo1)
    wsems = (wsem0, wsem1)

    def gather(t, buf):
        return pltpu.make_async_copy(
            tab_hbm.at[idx_v.at[lax.shift_right_logical(t, 3),
                                lax.bitwise_and(t, 7)]],
            buf, gsem)

    def step(t, carry):
        p = lax.rem(t, 2)

        def run(bufg, obufg, bufo, wsem):
            gather(t, bufg).wait()

            @pl.when(t + 1 < T)
            def _():
                gather(t + 1, obufg).start()

            @pl.when(t >= 2)
            def _():
                for dg in range(8):
                    pltpu.make_async_copy(
                        bufo.at[dg], out_hbm.at[t - 2, dg, wid], wsem).wait()

            posr = [pos_v[t, pl.ds(16 * c, 16)] for c in range(4)]
            dgc = [dgv + 2 * c for c in range(4)]

            def tok_body(tok, c2):
                tokv = jnp.full((16,), tok, dtype=jnp.int32)
                for c in range(4):
                    v = bufg[tok, pl.ds(16 * c, 16)] + posr[c]
                    plsc.store_scatter(bufo, [dgc[c], rv, tokv], v)
                return c2

            lax.fori_loop(0, 128, tok_body, 0)

            for dg in range(8):
                pltpu.async_copy(bufo.at[dg], out_hbm.at[t, dg, wid], wsem)

        @pl.when(p == 0)
        def _():
            run(bufg0, bufg1, bufo0, wsem0)

        @pl.when(p == 1)
        def _():
            run(bufg1, bufg0, bufo1, wsem1)

        return carry

    gather(0, bufg0).start()
    lax.fori_loop(0, T, step, 0)

    for t, (bufo, wsem) in ((T - 2, (bufo0, wsem0)), (T - 1, (bufo1, wsem1))):
        for dg in range(8):
            pltpu.make_async_copy(bufo.at[dg], out_hbm.at[t, dg, wid], wsem).wait()


@jax.jit
def kernel(x, token_table, pos_table):
    x4 = x.T.reshape(TG, 8, NW, 128).transpose(2, 0, 1, 3)

    mesh = plsc.VectorSubcoreMesh(core_axis_name="c", subcore_axis_name="s")
    k = functools.partial(
        pl.kernel,
        out_type=jax.ShapeDtypeStruct((T, 8, NW, 8, 128), jnp.float32),
        mesh=mesh,
        scratch_types=[
            pltpu.VMEM((TG, 8, 128), jnp.int32),
            pltpu.VMEM((T, D), jnp.float32),
            pltpu.VMEM((128, D), jnp.float32),
            pltpu.VMEM((128, D), jnp.float32),
            pltpu.VMEM((8, 8, 128), jnp.float32),
            pltpu.VMEM((8, 8, 128), jnp.float32),
            pltpu.SemaphoreType.DMA,
            pltpu.SemaphoreType.DMA,
            pltpu.SemaphoreType.DMA,
        ],
        compiler_params=pltpu.CompilerParams(
            use_tc_tiling_on_sc=False, needs_layout_passes=False),
    )(_body)
    out5 = k(x4, token_table, pos_table)
    return out5.transpose(2, 4, 0, 1, 3).reshape(B, T, D)


# speedup vs baseline: 1.5374x; 1.5374x over previous
import functools

import jax
import jax.numpy as jnp
from jax import lax
from jax.experimental import pallas as pl
from jax.experimental.pallas import tpu as pltpu
from jax.experimental.pallas import tpu_sc as plsc

B, T, D = 4096, 200, 64
NW = 32
TG = T // 8


def _body(x4_hbm, tab_hbm, pos_hbm, out_hbm, idx_v, pos_v, bufg0, bufg1,
          bufo0, bufo1, gsem, wsem0, wsem1):
    wid = lax.axis_index("s") * 2 + lax.axis_index("c")

    pltpu.sync_copy(x4_hbm.at[wid], idx_v)
    pltpu.sync_copy(pos_hbm, pos_v)

    k = lax.iota(jnp.int32, 16)
    rv = lax.bitwise_and(k, 7)
    dgv = lax.shift_right_logical(k, 3)

    bufgs = (bufg0, bufg1)
    bufos = (bufo0, bufo1)
    wsems = (wsem0, wsem1)

    def gather(t, buf):
        return pltpu.make_async_copy(
            tab_hbm.at[idx_v.at[lax.shift_right_logical(t, 3),
                                lax.bitwise_and(t, 7)]],
            buf, gsem)

    def step(t, carry):
        p = lax.rem(t, 2)

        def run(bufg, obufg, bufo, wsem):
            gather(t, bufg).wait()

            @pl.when(t + 1 < T)
            def _():
                gather(t + 1, obufg).start()

            @pl.when(t >= 2)
            def _():
                for dg in range(8):
                    pltpu.make_async_copy(
                        bufo.at[dg, pl.ds(0, 8), pl.ds(0, 128)],
                        out_hbm.at[t - 2, dg, wid], wsem).wait()

            posr = [pos_v[t, pl.ds(16 * c, 16)] for c in range(4)]
            dgc = [dgv + 2 * c for c in range(4)]

            def tok_body(tok, c2):
                tokv = jnp.full((16,), tok, dtype=jnp.int32)
                for c in range(4):
                    v = bufg[tok, pl.ds(16 * c, 16)] + posr[c]
                    plsc.store_scatter(bufo, [dgc[c], rv, tokv], v)
                return c2

            lax.fori_loop(0, 128, tok_body, 0)

            for dg in range(8):
                pltpu.async_copy(bufo.at[dg, pl.ds(0, 8), pl.ds(0, 128)],
                                 out_hbm.at[t, dg, wid], wsem)

        @pl.when(p == 0)
        def _():
            run(bufg0, bufg1, bufo0, wsem0)

        @pl.when(p == 1)
        def _():
            run(bufg1, bufg0, bufo1, wsem1)

        return carry

    gather(0, bufg0).start()
    lax.fori_loop(0, T, step, 0)

    for t, (bufo, wsem) in ((T - 2, (bufo0, wsem0)), (T - 1, (bufo1, wsem1))):
        for dg in range(8):
            pltpu.make_async_copy(bufo.at[dg, pl.ds(0, 8), pl.ds(0, 128)],
                                  out_hbm.at[t, dg, wid], wsem).wait()


@jax.jit
def kernel(x, token_table, pos_table):
    x4 = x.T.reshape(TG, 8, NW, 128).transpose(2, 0, 1, 3)

    mesh = plsc.VectorSubcoreMesh(core_axis_name="c", subcore_axis_name="s")
    k = functools.partial(
        pl.kernel,
        out_type=jax.ShapeDtypeStruct((T, 8, NW, 8, 128), jnp.float32),
        mesh=mesh,
        scratch_types=[
            pltpu.VMEM((TG, 8, 128), jnp.int32),
            pltpu.VMEM((T, D), jnp.float32),
            pltpu.VMEM((128, D), jnp.float32),
            pltpu.VMEM((128, D), jnp.float32),
            pltpu.VMEM((8, 8, 129), jnp.float32),
            pltpu.VMEM((8, 8, 129), jnp.float32),
            pltpu.SemaphoreType.DMA,
            pltpu.SemaphoreType.DMA,
            pltpu.SemaphoreType.DMA,
        ],
        compiler_params=pltpu.CompilerParams(
            use_tc_tiling_on_sc=False, needs_layout_passes=False),
    )(_body)
    out5 = k(x4, token_table, pos_table)
    return out5.transpose(2, 4, 0, 1, 3).reshape(B, T, D)
